# baseline (speedup 1.0000x reference)
import functools

import jax
import jax.numpy as jnp
from jax import lax
from jax.experimental import pallas as pl
from jax.experimental.pallas import tpu as pltpu
from jax.experimental.pallas import tpu_sc as plsc

_NUM_CORES = 2
_NUM_SUBCORES = 16
_NUM_WORKERS = _NUM_CORES * _NUM_SUBCORES


def kernel(x, row_embed):
    b = x.shape[0]
    w = x.shape[-2]
    d = row_embed.shape[-1]
    rows_per = w // _NUM_WORKERS

    mesh = plsc.VectorSubcoreMesh(core_axis_name="c", subcore_axis_name="s")

    @functools.partial(
        pl.kernel,
        mesh=mesh,
        out_type=jax.ShapeDtypeStruct((b * w, d), row_embed.dtype),
        scratch_types=[
            pltpu.VMEM((rows_per, d), row_embed.dtype),
            pltpu.VMEM_SHARED((_NUM_SUBCORES, rows_per, d), row_embed.dtype),
            pltpu.SemaphoreType.DMA,
            pltpu.SemaphoreType.DMA,
            pltpu.SemaphoreType.DMA,
        ],
    )
    def _bcast(emb_hbm, out_hbm, buf, shared, sem_r, sem_s, sem_w):
        cid = lax.axis_index("c")
        sid = lax.axis_index("s")
        wid = sid * _NUM_CORES + cid
        base = wid * rows_per
        s1 = pltpu.async_copy(emb_hbm.at[pl.ds(base, rows_per)], shared.at[sid], sem_s)
        pltpu.async_copy(emb_hbm.at[pl.ds(base, rows_per)], buf, sem_r).wait()
        writes = [
            pltpu.async_copy(buf, out_hbm.at[pl.ds(bb * w + base, rows_per)], sem_w)
            for bb in range(b - 1)
        ]
        s1.wait()
        writes.append(
            pltpu.async_copy(
                shared.at[sid], out_hbm.at[pl.ds((b - 1) * w + base, rows_per)], sem_w
            )
        )
        for c in writes:
            c.wait()

    return _bcast(row_embed).reshape(b, w, d)


# speedup vs baseline: 1.0689x; 1.0689x over previous
import functools

import jax
import jax.numpy as jnp
from jax import lax
from jax.experimental import pallas as pl
from jax.experimental.pallas import tpu as pltpu
from jax.experimental.pallas import tpu_sc as plsc

_NUM_CORES = 2
_NUM_SUBCORES = 16
_NUM_WORKERS = _NUM_CORES * _NUM_SUBCORES


def kernel(x, row_embed):
    b = x.shape[0]
    w = x.shape[-2]
    d = row_embed.shape[-1]
    rows_per = w // _NUM_WORKERS

    mesh = plsc.VectorSubcoreMesh(core_axis_name="c", subcore_axis_name="s")

    half = rows_per // 2

    @functools.partial(
        pl.kernel,
        mesh=mesh,
        out_type=jax.ShapeDtypeStruct((b * w, d), row_embed.dtype),
        scratch_types=[
            pltpu.VMEM((half, d), row_embed.dtype),
            pltpu.VMEM((half, d), row_embed.dtype),
            pltpu.SemaphoreType.DMA,
            pltpu.SemaphoreType.DMA,
            pltpu.SemaphoreType.DMA,
        ],
    )
    def _bcast(emb_hbm, out_hbm, buf0, buf1, sem_r0, sem_r1, sem_w):
        wid = lax.axis_index("s") * _NUM_CORES + lax.axis_index("c")
        base = wid * rows_per
        r0 = pltpu.async_copy(emb_hbm.at[pl.ds(base, half)], buf0, sem_r0)
        r1 = pltpu.async_copy(emb_hbm.at[pl.ds(base + half, half)], buf1, sem_r1)
        rot = lax.rem(wid, b)
        r0.wait()
        writes = []
        for k in range(b):
            bb = lax.rem(rot + k, b)
            writes.append(
                pltpu.async_copy(buf0, out_hbm.at[pl.ds(bb * w + base, half)], sem_w)
            )
        r1.wait()
        for k in range(b):
            bb = lax.rem(rot + k, b)
            writes.append(
                pltpu.async_copy(
                    buf1, out_hbm.at[pl.ds(bb * w + base + half, half)], sem_w
                )
            )
        for c in writes:
            c.wait()

    return _bcast(row_embed).reshape(b, w, d)
